# Initial kernel scaffold; baseline (speedup 1.0000x reference)
import jax
import jax.numpy as jnp
from jax.experimental import pallas as pl


def kernel(batch_tokens, lengths, table):
    raise NotImplementedError("write your pallas kernel here")


# speedup vs baseline: 1.0577x; 1.0577x over previous
import functools

import jax
import jax.numpy as jnp
from jax import lax
from jax.experimental import pallas as pl
from jax.experimental.pallas import tpu as pltpu
from jax.experimental.pallas import tpu_sc as plsc

_B = 1024
_L = 50
_D = 768
_NC = 2
_NS = 16
_NW = _NC * _NS
_RPW = _B * _L // _NW
_SPW = _B // _NW
_LANES = 16
_DV = _D // _LANES
_CH = 64
_NCHUNK = _RPW // _CH
_REP = 8


def _make_kernel():
    mesh = plsc.VectorSubcoreMesh(core_axis_name="c", subcore_axis_name="s")

    @functools.partial(
        pl.kernel,
        mesh=mesh,
        out_type=[
            jax.ShapeDtypeStruct((_B * _L, _D), jnp.float32),
            jax.ShapeDtypeStruct((_B * _L,), jnp.int32),
        ],
        scratch_types=[
            pltpu.VMEM((_RPW,), jnp.int32),
            pltpu.VMEM(((_SPW + 1) * _REP,), jnp.int32),
            pltpu.VMEM((_RPW,), jnp.int32),
            pltpu.VMEM((_CH, _D), jnp.float32),
            pltpu.SemaphoreType.DMA,
        ],
    )
    def k(tok_hbm, lrep_hbm, table_hbm, out_hbm, mask_hbm,
          tok_v, len_v, mask_v, rows_v, sem):
        wid = lax.axis_index("s") * _NC + lax.axis_index("c")
        row0 = wid * _RPW
        seq0 = wid * _SPW
        pltpu.sync_copy(tok_hbm.at[pl.ds(row0, _RPW)], tok_v)
        pltpu.sync_copy(lrep_hbm.at[pl.ds(seq0 * _REP, (_SPW + 1) * _REP)],
                        len_v)

        zeros = jnp.zeros((_LANES,), jnp.float32)
        lane_iota = lax.iota(jnp.int32, _LANES)

        def len_at(s):
            return len_v[pl.ds(s * _REP, _LANES)][0]

        def mask_vec(n, carry):
            p0 = n * _LANES
            a = p0 // _L
            j0 = p0 - a * _L
            ln_a = jnp.zeros((_LANES,), jnp.int32) + len_at(a)
            ln_b = jnp.zeros((_LANES,), jnp.int32) + len_at(a + 1)
            j = j0 + lane_iota
            in_a = j < _L
            lv = jnp.where(in_a, ln_a, ln_b)
            jj = jnp.where(in_a, j, j - _L)
            mask_v[pl.ds(p0, _LANES)] = jnp.where(
                jj < lv, jnp.ones((_LANES,), jnp.int32),
                jnp.zeros((_LANES,), jnp.int32))
            return carry
        lax.fori_loop(0, _RPW // _LANES, mask_vec, 0)
        pltpu.sync_copy(mask_v, mask_hbm.at[pl.ds(row0, _RPW)])

        def per_chunk(c, carry):
            base = c * _CH
            pltpu.async_copy(
                table_hbm.at[tok_v.at[pl.ds(base, _CH)]], rows_v, sem
            ).wait()

            def zrow(r, c2):
                g = base + r
                s = g // _L
                j = g - s * _L
                ln = len_at(s)

                @pl.when(j >= ln)
                def _():
                    for v in range(_DV):
                        rows_v[r, pl.ds(v * _LANES, _LANES)] = zeros
                return c2
            lax.fori_loop(0, _CH, zrow, 0)

            pltpu.sync_copy(rows_v, out_hbm.at[pl.ds(row0 + base, _CH)])
            return carry

        lax.fori_loop(0, _NCHUNK, per_chunk, 0)

    return k


_sc_kernel = _make_kernel()


def kernel(batch_tokens, lengths, table):
    tok = batch_tokens.reshape(-1).astype(jnp.int32)
    lrep = jnp.concatenate(
        [jnp.repeat(lengths.astype(jnp.int32), _REP),
         jnp.zeros((_REP,), jnp.int32)]
    )
    out_flat, mask_i = _sc_kernel(tok, lrep, table)
    embs = out_flat.reshape(_B, _L, _D)
    mask = mask_i.reshape(_B, _L).astype(bool)
    return embs, mask


# speedup vs baseline: 1.2280x; 1.1610x over previous
import functools

import jax
import jax.numpy as jnp
from jax import lax
from jax.experimental import pallas as pl
from jax.experimental.pallas import tpu as pltpu
from jax.experimental.pallas import tpu_sc as plsc

_B = 1024
_L = 50
_D = 768
_NC = 2
_NS = 16
_NW = _NC * _NS
_RPW = _B * _L // _NW
_SPW = _B // _NW
_LANES = 16
_DV = _D // _LANES
_CH = 40
_NCHUNK = _RPW // _CH
_REP = 8


def _make_kernel():
    mesh = plsc.VectorSubcoreMesh(core_axis_name="c", subcore_axis_name="s")

    @functools.partial(
        pl.kernel,
        mesh=mesh,
        out_type=[
            jax.ShapeDtypeStruct((_B * _L, _D), jnp.float32),
            jax.ShapeDtypeStruct((_B * _L,), jnp.int32),
        ],
        scratch_types=[
            pltpu.VMEM((_RPW,), jnp.int32),
            pltpu.VMEM(((_SPW + 1) * _REP,), jnp.int32),
            pltpu.VMEM((_RPW,), jnp.int32),
            pltpu.VMEM((_CH, _D), jnp.float32),
            pltpu.VMEM((_CH, _D), jnp.float32),
            pltpu.SemaphoreType.DMA,
            pltpu.SemaphoreType.DMA,
            pltpu.SemaphoreType.DMA,
            pltpu.SemaphoreType.DMA,
        ],
    )
    def k(tok_hbm, lrep_hbm, table_hbm, out_hbm, mask_hbm,
          tok_v, len_v, mask_v, rows0_v, rows1_v, g0, g1, w0, w1):
        rows = (rows0_v, rows1_v)
        gsem = (g0, g1)
        wsem = (w0, w1)

        wid = lax.axis_index("s") * _NC + lax.axis_index("c")
        row0 = wid * _RPW
        seq0 = wid * _SPW
        pltpu.sync_copy(tok_hbm.at[pl.ds(row0, _RPW)], tok_v)
        pltpu.sync_copy(lrep_hbm.at[pl.ds(seq0 * _REP, (_SPW + 1) * _REP)],
                        len_v)

        zeros = jnp.zeros((_LANES,), jnp.float32)
        lane_iota = lax.iota(jnp.int32, _LANES)

        def gather(c, b):
            return pltpu.make_async_copy(
                table_hbm.at[tok_v.at[pl.ds(c * _CH, _CH)]], rows[b], gsem[b])

        def write(c, b):
            return pltpu.make_async_copy(
                rows[b], out_hbm.at[pl.ds(row0 + c * _CH, _CH)], wsem[b])

        gather(0, 0).start()
        gather(1, 1).start()

        def len_at(s):
            return len_v[pl.ds(s * _REP, _LANES)][0]

        def mask_vec(n, carry):
            p0 = n * _LANES
            a = p0 // _L
            j0 = p0 - a * _L
            ln_a = jnp.zeros((_LANES,), jnp.int32) + len_at(a)
            ln_b = jnp.zeros((_LANES,), jnp.int32) + len_at(a + 1)
            j = j0 + lane_iota
            in_a = j < _L
            lv = jnp.where(in_a, ln_a, ln_b)
            jj = jnp.where(in_a, j, j - _L)
            mask_v[pl.ds(p0, _LANES)] = jnp.where(
                jj < lv, jnp.ones((_LANES,), jnp.int32),
                jnp.zeros((_LANES,), jnp.int32))
            return carry
        lax.fori_loop(0, _RPW // _LANES, mask_vec, 0)
        pltpu.sync_copy(mask_v, mask_hbm.at[pl.ds(row0, _RPW)])

        def process(c, b):
            p0 = c * _CH
            a = p0 // _L
            j0 = p0 - a * _L
            ln_a = len_at(a)
            ln_b = len_at(a + 1)
            m = _L - j0
            mm = jnp.minimum(m, _CH)

            def zrow(r, c2):
                for v in range(_DV):
                    rows[b][r, pl.ds(v * _LANES, _LANES)] = zeros
                return c2

            za = jnp.clip(ln_a - j0, 0, mm)
            lax.fori_loop(za, mm, zrow, 0)
            zb = jnp.minimum(m + ln_b, _CH)
            lax.fori_loop(zb, _CH, zrow, 0)

        def step(c, b):
            gather(c, b).wait()
            process(c, b)
            write(c, b).start()
            o = 1 - b

            @pl.when(jnp.logical_and(c >= 1, c + 1 < _NCHUNK))
            def _():
                write(c - 1, o).wait()
                gather(c + 1, o).start()

        def pair(i, carry):
            step(i * 2, 0)
            step(i * 2 + 1, 1)
            return carry
        lax.fori_loop(0, _NCHUNK // 2, pair, 0)

        write(_NCHUNK - 2, (_NCHUNK - 2) % 2).wait()
        write(_NCHUNK - 1, (_NCHUNK - 1) % 2).wait()

    return k


_sc_kernel = _make_kernel()


def kernel(batch_tokens, lengths, table):
    tok = batch_tokens.reshape(-1).astype(jnp.int32)
    lrep = jnp.concatenate(
        [jnp.repeat(lengths.astype(jnp.int32), _REP),
         jnp.zeros((_REP,), jnp.int32)]
    )
    out_flat, mask_i = _sc_kernel(tok, lrep, table)
    embs = out_flat.reshape(_B, _L, _D)
    mask = mask_i.reshape(_B, _L).astype(bool)
    return embs, mask


# speedup vs baseline: 3.0274x; 2.4654x over previous
import functools

import jax
import jax.numpy as jnp
from jax import lax
from jax.experimental import pallas as pl
from jax.experimental.pallas import tpu as pltpu
from jax.experimental.pallas import tpu_sc as plsc

_B = 1024
_L = 50
_D = 768
_NC = 2
_NS = 16
_NW = _NC * _NS
_RPW = _B * _L // _NW
_LANES = 16
_DV = _D // _LANES
_CH = 64
_NCHUNK = _RPW // _CH
_GRP = _CH // _LANES


def _make_kernel():
    mesh = plsc.VectorSubcoreMesh(core_axis_name="c", subcore_axis_name="s")

    @functools.partial(
        pl.kernel,
        mesh=mesh,
        out_type=[
            jax.ShapeDtypeStruct((_B * _L, _D), jnp.float32),
            jax.ShapeDtypeStruct((_B * _L,), jnp.int32),
        ],
        scratch_types=[
            pltpu.VMEM((_RPW,), jnp.int32),
            pltpu.VMEM((_B,), jnp.int32),
            pltpu.VMEM((_RPW,), jnp.int32),
            pltpu.VMEM((_CH, _D), jnp.float32),
            pltpu.VMEM((_CH, _D), jnp.float32),
            pltpu.SemaphoreType.DMA,
            pltpu.SemaphoreType.DMA,
            pltpu.SemaphoreType.DMA,
            pltpu.SemaphoreType.DMA,
        ],
    )
    def k(tok_hbm, len_hbm, table_hbm, out_hbm, mask_hbm,
          tok_v, len_v, mask_v, rows0_v, rows1_v, g0, g1, w0, w1):
        rows = (rows0_v, rows1_v)
        gsem = (g0, g1)
        wsem = (w0, w1)

        wid = lax.axis_index("s") * _NC + lax.axis_index("c")
        row0 = wid * _RPW
        pltpu.sync_copy(tok_hbm.at[pl.ds(row0, _RPW)], tok_v)
        pltpu.sync_copy(len_hbm, len_v)

        zeros = jnp.zeros((_LANES,), jnp.float32)

        def gather(c, b):
            return pltpu.make_async_copy(
                table_hbm.at[tok_v.at[pl.ds(c * _CH, _CH)]], rows[b], gsem[b])

        def write(c, b):
            return pltpu.make_async_copy(
                rows[b], out_hbm.at[pl.ds(row0 + c * _CH, _CH)], wsem[b])

        gather(0, 0).start()
        gather(1, 1).start()

        def mask_vec(n, carry):
            p0 = row0 + n * _LANES
            j = p0 // _B
            i0 = p0 - j * _B
            lv = len_v[pl.ds(i0, _LANES)]
            jv = jnp.zeros((_LANES,), jnp.int32) + j
            mask_v[pl.ds(n * _LANES, _LANES)] = jnp.where(
                jv < lv, jnp.ones((_LANES,), jnp.int32),
                jnp.zeros((_LANES,), jnp.int32))
            return carry
        lax.fori_loop(0, _RPW // _LANES, mask_vec, 0)
        pltpu.sync_copy(mask_v, mask_hbm.at[pl.ds(row0, _RPW)])

        def process(c, b):
            p0 = row0 + c * _CH
            j = p0 // _B
            i0 = p0 - j * _B

            def grp(g, carry):
                lv = len_v[pl.ds(i0 + g * _LANES, _LANES)]
                for r in range(_LANES):
                    @pl.when(j >= lv[r])
                    def _():
                        row = g * _LANES + r
                        for v in range(_DV):
                            rows[b][row, pl.ds(v * _LANES, _LANES)] = zeros
                return carry
            lax.fori_loop(0, _GRP, grp, 0)

        def step(c, b):
            gather(c, b).wait()
            process(c, b)
            write(c, b).start()
            o = 1 - b

            @pl.when(jnp.logical_and(c >= 1, c + 1 < _NCHUNK))
            def _():
                write(c - 1, o).wait()
                gather(jnp.minimum(c + 1, _NCHUNK - 1), o).start()

        def pair(t, carry):
            step(t * 2, 0)
            step(t * 2 + 1, 1)
            return carry
        lax.fori_loop(0, _NCHUNK // 2, pair, 0)
        step(_NCHUNK - 1, (_NCHUNK - 1) % 2)

        write(_NCHUNK - 2, (_NCHUNK - 2) % 2).wait()
        write(_NCHUNK - 1, (_NCHUNK - 1) % 2).wait()

    return k


_sc_kernel = _make_kernel()


def kernel(batch_tokens, lengths, table):
    tok = batch_tokens.astype(jnp.int32).T.reshape(-1)
    out_flat, mask_i = _sc_kernel(tok, lengths.astype(jnp.int32), table)
    embs = out_flat.reshape(_L, _B, _D).transpose(1, 0, 2)
    mask = (mask_i.reshape(_L, _B) != 0).T
    return embs, mask


# speedup vs baseline: 3.8833x; 1.2827x over previous
import functools

import jax
import jax.numpy as jnp
from jax import lax
from jax.experimental import pallas as pl
from jax.experimental.pallas import tpu as pltpu
from jax.experimental.pallas import tpu_sc as plsc

_B = 1024
_L = 50
_D = 768
_NC = 2
_NS = 16
_NW = _NC * _NS
_RPW = _B * _L // _NW
_LANES = 16
_DV = _D // _LANES
_CH = 64
_NCHUNK = _RPW // _CH
_GSC = _CH // _LANES
_REP = 8


def _make_kernel():
    mesh = plsc.VectorSubcoreMesh(core_axis_name="c", subcore_axis_name="s")

    @functools.partial(
        pl.kernel,
        mesh=mesh,
        out_type=[
            jax.ShapeDtypeStruct((_B * _L, _D), jnp.float32),
            jax.ShapeDtypeStruct((_B * _L,), jnp.int32),
        ],
        scratch_types=[
            pltpu.VMEM((_RPW,), jnp.int32),
            pltpu.VMEM((_B,), jnp.int32),
            pltpu.VMEM((_L * _REP + _REP,), jnp.int32),
            pltpu.VMEM((_RPW,), jnp.int32),
            pltpu.VMEM((_RPW,), jnp.int32),
            pltpu.VMEM((_CH, _D), jnp.float32),
            pltpu.VMEM((_CH, _D), jnp.float32),
            pltpu.VMEM((_LANES, _D), jnp.float32),
            pltpu.SemaphoreType.DMA,
            pltpu.SemaphoreType.DMA,
            pltpu.SemaphoreType.DMA,
            pltpu.SemaphoreType.DMA,
        ],
    )
    def k(tok_hbm, len_hbm, cnt_hbm, dst_hbm, table_hbm, out_hbm, mask_hbm,
          tok_v, len_v, cnt_v, dst_v, mask_v, rows0_v, rows1_v, zrows_v,
          g0, g1, w0, w1):
        rows = (rows0_v, rows1_v)
        gsem = (g0, g1)
        wsem = (w0, w1)

        wid = lax.axis_index("s") * _NC + lax.axis_index("c")
        row0 = wid * _RPW
        pltpu.sync_copy(tok_hbm.at[pl.ds(row0, _RPW)], tok_v)
        pltpu.sync_copy(len_hbm, len_v)
        pltpu.sync_copy(cnt_hbm, cnt_v)
        pltpu.sync_copy(dst_hbm.at[pl.ds(row0, _RPW)], dst_v)

        zeros = jnp.zeros((_LANES,), jnp.float32)

        def zfill(r, carry):
            for v in range(_DV):
                zrows_v[r, pl.ds(v * _LANES, _LANES)] = zeros
            return carry
        lax.fori_loop(0, _LANES, zfill, 0)

        def cnt_at(j):
            return cnt_v[pl.ds(j * _REP, _LANES)][0]

        def nvalid(c):
            p0 = row0 + c * _CH
            j = p0 // _B
            k0 = p0 - j * _B
            return jnp.clip(cnt_at(j) - k0, 0, _CH)

        def gather(c, b):
            return pltpu.make_async_copy(
                table_hbm.at[tok_v.at[pl.ds(c * _CH, _CH)]], rows[b], gsem[b])

        def scatter(c, q, src, sem):
            ivec = dst_v[pl.ds((c * _GSC + q) * _LANES, _LANES)]
            return pltpu.make_async_copy(src, out_hbm.at[ivec], sem)

        @pl.when(nvalid(0) > 0)
        def _():
            gather(0, 0).start()

        @pl.when(nvalid(1) > 0)
        def _():
            gather(1, 1).start()

        def mask_vec(n, carry):
            p0 = row0 + n * _LANES
            j = p0 // _B
            i0 = p0 - j * _B
            lv = len_v[pl.ds(i0, _LANES)]
            jv = jnp.zeros((_LANES,), jnp.int32) + j
            mask_v[pl.ds(n * _LANES, _LANES)] = jnp.where(
                jv < lv, jnp.ones((_LANES,), jnp.int32),
                jnp.zeros((_LANES,), jnp.int32))
            return carry
        lax.fori_loop(0, _RPW // _LANES, mask_vec, 0)
        pltpu.sync_copy(mask_v, mask_hbm.at[pl.ds(row0, _RPW)])

        def emit_writes(c, b):
            nv = nvalid(c)

            @pl.when(nv > 0)
            def _():
                def zrow(r, carry):
                    for v in range(_DV):
                        rows[b][r, pl.ds(v * _LANES, _LANES)] = zeros
                    return carry
                lax.fori_loop(nv, _CH, zrow, 0)
                for q in range(_GSC):
                    scatter(c, q, rows[b].at[pl.ds(q * _LANES, _LANES)],
                            wsem[b]).start()

            @pl.when(nv == 0)
            def _():
                for q in range(_GSC):
                    scatter(c, q, zrows_v, wsem[b]).start()

        def wait_writes(c, b):
            for q in range(_GSC):
                scatter(c, q, zrows_v, wsem[b]).wait()

        def step(c, b):
            @pl.when(nvalid(c) > 0)
            def _():
                gather(c, b).wait()
            o = 1 - b

            @pl.when(jnp.logical_and(c >= 1, c + 1 < _NCHUNK))
            def _():
                wait_writes(c - 1, o)
                nc = jnp.minimum(c + 1, _NCHUNK - 1)

                @pl.when(nvalid(nc) > 0)
                def _():
                    gather(nc, o).start()
            emit_writes(c, b)

        def pair(t, carry):
            step(t * 2, 0)
            step(t * 2 + 1, 1)
            return carry
        lax.fori_loop(0, _NCHUNK // 2, pair, 0)
        step(_NCHUNK - 1, (_NCHUNK - 1) % 2)

        wait_writes(_NCHUNK - 2, (_NCHUNK - 2) % 2)
        wait_writes(_NCHUNK - 1, (_NCHUNK - 1) % 2)

    return k


_sc_kernel = _make_kernel()


def kernel(batch_tokens, lengths, table):
    lengths = lengths.astype(jnp.int32)
    perm = jnp.argsort(-lengths).astype(jnp.int32)
    tok = batch_tokens.astype(jnp.int32)[perm].T.reshape(-1)
    cnt = jnp.sum(lengths[None, :] > jnp.arange(_L, dtype=jnp.int32)[:, None],
                  axis=1, dtype=jnp.int32)
    cnt_rep = jnp.concatenate(
        [jnp.repeat(cnt, _REP), jnp.zeros((_REP,), jnp.int32)])
    dst = (jnp.arange(_L, dtype=jnp.int32)[:, None] * _B + perm[None, :]
           ).reshape(-1)
    out_flat, mask_i = _sc_kernel(tok, lengths, cnt_rep, dst, table)
    embs = out_flat.reshape(_L, _B, _D).transpose(1, 0, 2)
    mask = (mask_i.reshape(_L, _B) != 0).T
    return embs, mask
